# Initial kernel scaffold; baseline (speedup 1.0000x reference)
import jax
import jax.numpy as jnp
from jax.experimental import pallas as pl


def kernel(x, edge_index, edge_attr, message, W_l, b_l, W_r, b_r, W_e, att, bias, W_fc, b_fc):
    raise NotImplementedError("write your pallas kernel here")


# speedup vs baseline: 188.9665x; 188.9665x over previous
import functools

import jax
import jax.numpy as jnp
from jax import lax
from jax.experimental import pallas as pl
from jax.experimental.pallas import tpu as pltpu
from jax.experimental.pallas import tpu_sc as plsc

N = 50000
E = 800000
H = 2
C = 32
HC = H * C
HID = 128
B = 32

NC = 2
NS = 16
NW = NC * NS
L = 16

EPW = 25088
E_PAD = NW * EPW
E_ROWS = E_PAD // 128
RPW = EPW // 128
G_ITERS = 49

N_PAD = 50176
N_ROWS = N_PAD // 128
SLC = N_PAD // NS

DEAD = N

EBLK = 128
NBLK = 8
GRID = E_ROWS // EBLK

def _mesh():
    return plsc.VectorSubcoreMesh(
        core_axis_name="c", subcore_axis_name="s",
        num_cores=NC, num_subcores=NS)


@functools.cache
def _build_sc_pass1():
    return functools.partial(
        pl.kernel,
        out_type=(
            jax.ShapeDtypeStruct((NC * N_PAD,), jnp.float32),
            jax.ShapeDtypeStruct((NC * N_PAD,), jnp.float32),
            jax.ShapeDtypeStruct((E_ROWS, 128), jnp.float32),
            jax.ShapeDtypeStruct((E_ROWS, 128), jnp.float32),
        ),
        mesh=_mesh(),
        compiler_params=pltpu.CompilerParams(needs_layout_passes=False),
        scratch_types=[
            pltpu.VMEM((N_PAD,), jnp.float32),
            pltpu.VMEM((4, 128), jnp.int32),
            pltpu.VMEM((4, 128), jnp.int32),
            pltpu.VMEM((4, 128), jnp.float32),
            pltpu.VMEM((4, 128), jnp.float32),
            pltpu.VMEM((4, 128), jnp.float32),
            pltpu.VMEM((128,), jnp.float32),
            pltpu.VMEM((SLC,), jnp.float32),
            pltpu.VMEM_SHARED((N_PAD,), jnp.float32),
            pltpu.VMEM_SHARED((N_PAD,), jnp.float32),
        ],
    )(_sc_pass1_body)


def _zero_vmem(buf, n):
    def zb(i, carry):
        buf[pl.ds(i * L, L)] = jnp.zeros((L,), jnp.float32)
        return carry
    lax.fori_loop(0, n // L, zb, 0)


def _sc_pass1_body(x_hbm, src_hbm, dst_hbm, ea_hbm,
                   cnt_out, sum_out, xs_out, xd_out,
                   x_v, srcb, dstb, eab, xsb, xdb, onesb, zbuf,
                   cnt_acc, sum_acc):
    cid = lax.axis_index("c")
    sid = lax.axis_index("s")
    wid = cid * NS + sid
    for k in range(8):
        onesb[pl.ds(k * L, L)] = jnp.ones((L,), jnp.float32)
    _zero_vmem(zbuf, SLC)
    pltpu.sync_copy(x_hbm, x_v)
    off = sid * SLC
    pltpu.sync_copy(zbuf, cnt_acc.at[pl.ds(off, SLC)])
    pltpu.sync_copy(zbuf, sum_acc.at[pl.ds(off, SLC)])
    plsc.subcore_barrier()

    rbase = wid * RPW

    def body(g, carry):
        rb = rbase + g * 4
        pltpu.sync_copy(src_hbm.at[pl.ds(rb, 4)], srcb)
        pltpu.sync_copy(dst_hbm.at[pl.ds(rb, 4)], dstb)
        pltpu.sync_copy(ea_hbm.at[pl.ds(rb, 4)], eab)
        for j in range(4):
            for k in range(8):
                sl = pl.ds(k * L, L)
                xsb[j, sl] = plsc.load_gather(x_v, [srcb[j, sl]])
                xdb[j, sl] = plsc.load_gather(x_v, [dstb[j, sl]])
        pltpu.sync_copy(xsb, xs_out.at[pl.ds(rb, 4)])
        pltpu.sync_copy(xdb, xd_out.at[pl.ds(rb, 4)])
        for j in range(4):
            pltpu.sync_copy(onesb, cnt_acc.at[dstb.at[j]], add=True)
            pltpu.sync_copy(eab.at[j], sum_acc.at[dstb.at[j]], add=True)
        return carry

    lax.fori_loop(0, G_ITERS, body, 0)
    plsc.subcore_barrier()
    pltpu.sync_copy(cnt_acc.at[pl.ds(off, SLC)], zbuf)
    pltpu.sync_copy(zbuf, cnt_out.at[pl.ds(cid * N_PAD + off, SLC)])
    pltpu.sync_copy(sum_acc.at[pl.ds(off, SLC)], zbuf)
    pltpu.sync_copy(zbuf, sum_out.at[pl.ds(cid * N_PAD + off, SLC)])


@functools.cache
def _build_sc_pass2():
    return functools.partial(
        pl.kernel,
        out_type=tuple(jax.ShapeDtypeStruct((NC * N_PAD,), jnp.float32)
                       for _ in range(4)),
        mesh=_mesh(),
        compiler_params=pltpu.CompilerParams(needs_layout_passes=False),
        scratch_types=[
            pltpu.VMEM((4, 128), jnp.int32),
            pltpu.VMEM((4, 128), jnp.float32),
            pltpu.VMEM((4, 128), jnp.float32),
            pltpu.VMEM((4, 128), jnp.float32),
            pltpu.VMEM((4, 128), jnp.float32),
            pltpu.VMEM((SLC,), jnp.float32),
            pltpu.VMEM_SHARED((N_PAD,), jnp.float32),
            pltpu.VMEM_SHARED((N_PAD,), jnp.float32),
            pltpu.VMEM_SHARED((N_PAD,), jnp.float32),
            pltpu.VMEM_SHARED((N_PAD,), jnp.float32),
        ],
    )(_sc_pass2_body)


def _sc_pass2_body(dst_hbm, p0_hbm, p1_hbm, q0_hbm, q1_hbm,
                   o0, o1, o2, o3,
                   dstb, p0b, p1b, q0b, q1b, zbuf, a0, a1, a2, a3):
    cid = lax.axis_index("c")
    sid = lax.axis_index("s")
    wid = cid * NS + sid
    off = sid * SLC
    _zero_vmem(zbuf, SLC)
    for a in (a0, a1, a2, a3):
        pltpu.sync_copy(zbuf, a.at[pl.ds(off, SLC)])
    plsc.subcore_barrier()

    rbase = wid * RPW

    def body(g, carry):
        rb = rbase + g * 4
        pltpu.sync_copy(dst_hbm.at[pl.ds(rb, 4)], dstb)
        pltpu.sync_copy(p0_hbm.at[pl.ds(rb, 4)], p0b)
        pltpu.sync_copy(p1_hbm.at[pl.ds(rb, 4)], p1b)
        pltpu.sync_copy(q0_hbm.at[pl.ds(rb, 4)], q0b)
        pltpu.sync_copy(q1_hbm.at[pl.ds(rb, 4)], q1b)
        for j in range(4):
            idx = dstb.at[j]
            pltpu.sync_copy(p0b.at[j], a0.at[idx], add=True)
            pltpu.sync_copy(p1b.at[j], a1.at[idx], add=True)
            pltpu.sync_copy(q0b.at[j], a2.at[idx], add=True)
            pltpu.sync_copy(q1b.at[j], a3.at[idx], add=True)
        return carry

    lax.fori_loop(0, G_ITERS, body, 0)
    plsc.subcore_barrier()
    for o, a in zip((o0, o1, o2, o3), (a0, a1, a2, a3)):
        pltpu.sync_copy(a.at[pl.ds(off, SLC)], zbuf)
        pltpu.sync_copy(zbuf, o.at[pl.ds(cid * N_PAD + off, SLC)])


def _tc_edge_body(xs_r, xd_r, ea_r, x_r, cnt_r, sum_r,
                  wl_r, wr_r, we_r, bb_r, att_r,
                  p0_r, p1_r, q0_r, q1_r, den0_r, num0_r):
    xs = xs_r[...]
    xd = xd_r[...]
    ea = ea_r[...]
    l0 = jnp.zeros_like(xs)
    l1 = jnp.zeros_like(xs)
    for c in range(HC):
        m = xs * wl_r[0, c] + xd * wr_r[0, c] + ea * we_r[0, c] + bb_r[0, c]
        m = jnp.maximum(m, 0.2 * m)
        if c < C:
            l0 = l0 + att_r[0, c] * m
        else:
            l1 = l1 + att_r[0, c] * m
    p0 = jnp.exp(l0)
    p1 = jnp.exp(l1)
    p0_r[...] = p0
    p1_r[...] = p1
    q0_r[...] = p0 * xs
    q1_r[...] = p1 * xs

    xn = x_r[...]
    cnt = cnt_r[0] + cnt_r[1]
    sume = sum_r[0] + sum_r[1]
    la = sume / jnp.maximum(cnt, 1.0)
    s0 = jnp.zeros_like(xn)
    s1 = jnp.zeros_like(xn)
    for c in range(HC):
        m = xn * (wl_r[0, c] + wr_r[0, c]) + la * we_r[0, c] + bb_r[0, c]
        m = jnp.maximum(m, 0.2 * m)
        if c < C:
            s0 = s0 + att_r[0, c] * m
        else:
            s1 = s1 + att_r[0, c] * m
    e0 = jnp.exp(s0)
    e1 = jnp.exp(s1)
    den0_r[0] = e0
    den0_r[1] = e1
    num0_r[0] = e0 * xn
    num0_r[1] = e1 * xn


def _tc_pass(xs2d, xd2d, ea2d, x2d, cnt2d, sum2d, wl, wr, we, bb, att_f):
    espec = pl.BlockSpec((EBLK, 128), lambda i: (i, 0))
    nspec = pl.BlockSpec((NBLK, 128), lambda i: (i, 0))
    hspec = pl.BlockSpec((2, NBLK, 128), lambda i: (0, i, 0))
    sspec = pl.BlockSpec(memory_space=pltpu.SMEM)
    return pl.pallas_call(
        _tc_edge_body,
        grid=(GRID,),
        in_specs=[espec, espec, espec, nspec, hspec, hspec,
                  sspec, sspec, sspec, sspec, sspec],
        out_specs=[espec, espec, espec, espec, hspec, hspec],
        out_shape=[
            jax.ShapeDtypeStruct((E_ROWS, 128), jnp.float32),
            jax.ShapeDtypeStruct((E_ROWS, 128), jnp.float32),
            jax.ShapeDtypeStruct((E_ROWS, 128), jnp.float32),
            jax.ShapeDtypeStruct((E_ROWS, 128), jnp.float32),
            jax.ShapeDtypeStruct((2, N_ROWS, 128), jnp.float32),
            jax.ShapeDtypeStruct((2, N_ROWS, 128), jnp.float32),
        ],
    )(xs2d, xd2d, ea2d, x2d, cnt2d, sum2d, wl, wr, we, bb, att_f)


def _tc_final_body(c0_r, c1_r, c2_r, c3_r, den0_r, num0_r,
                   msg_r, wfc_r, bfc_r, wl_r, wb_r, out_r):
    me = lax.dot_general(msg_r[...], wfc_r[...],
                         (((1,), (1,)), ((), ())),
                         preferred_element_type=jnp.float32)
    me = me + bfc_r[...]
    mw = me * wl_r[...]
    u0 = jnp.sum(mw[:, :C], axis=1, keepdims=True)
    u1 = jnp.sum(mw[:, C:], axis=1, keepdims=True)
    w = jnp.sum(me * wb_r[...], axis=1, keepdims=True)

    den0_v = c0_r[0] + c0_r[1] + den0_r[0]
    den1_v = c1_r[0] + c1_r[1] + den0_r[1]
    num0_v = c2_r[0] + c2_r[1] + num0_r[0]
    num1_v = c3_r[0] + c3_r[1] + num0_r[1]
    sx0 = num0_v / den0_v
    sx1 = num1_v / den1_v
    for g in range(NBLK):
        d = u0 * sx0[g:g + 1, :] + u1 * sx1[g:g + 1, :] + w
        mx = jnp.max(d, axis=0, keepdims=True)
        ex = jnp.exp(d - mx)
        sm = jnp.sum(ex, axis=0, keepdims=True)
        out_r[pl.ds(g * 128, 128), :] = (ex / sm).T


def _tc_final(c0, c1, c2, c3, den0, num0, msg, wfc, bfc, wl, wb):
    hspec = pl.BlockSpec((2, NBLK, 128), lambda i: (0, i, 0))
    wspec = lambda shape: pl.BlockSpec(shape, lambda i: tuple(0 for _ in shape))
    return pl.pallas_call(
        _tc_final_body,
        grid=(GRID,),
        in_specs=[hspec, hspec, hspec, hspec, hspec, hspec,
                  wspec((B, HID)), wspec((HC, HID)), wspec((1, HC)),
                  wspec((1, HC)), wspec((1, HC))],
        out_specs=pl.BlockSpec((NBLK * 128, B), lambda i: (i, 0)),
        out_shape=jax.ShapeDtypeStruct((N_PAD, B), jnp.float32),
    )(c0, c1, c2, c3, den0, num0, msg, wfc, bfc, wl, wb)


def kernel(x, edge_index, edge_attr, message,
           W_l, b_l, W_r, b_r, W_e, att, bias, W_fc, b_fc):
    src = edge_index[0]
    dst = edge_index[1]
    ea = edge_attr[:, 0]
    pad = E_PAD - E
    src2d = jnp.pad(src, (0, pad)).reshape(E_ROWS, 128)
    dst2d = jnp.pad(dst, (0, pad), constant_values=DEAD).reshape(E_ROWS, 128)
    ea2d = jnp.pad(ea, (0, pad)).reshape(E_ROWS, 128)
    x_flat = jnp.pad(x[:, 0], (0, N_PAD - N))
    x2d = x_flat.reshape(N_ROWS, 128)

    cnt1, sum1, xs2d, xd2d = _build_sc_pass1()(x_flat, src2d, dst2d, ea2d)
    cnt2d = cnt1.reshape(NC, N_ROWS, 128)
    sum2d = sum1.reshape(NC, N_ROWS, 128)

    bb = (b_l + b_r).reshape(1, HC)
    att_f = att.reshape(1, HC)
    p0, p1, q0, q1, den0, num0 = _tc_pass(
        xs2d, xd2d, ea2d, x2d, cnt2d, sum2d, W_l, W_r, W_e, bb, att_f)

    accs = _build_sc_pass2()(dst2d, p0, p1, q0, q1)
    c0, c1, c2, c3 = (a.reshape(NC, N_ROWS, 128) for a in accs)

    wb = (b_l + bias).reshape(1, HC)
    outp = _tc_final(c0, c1, c2, c3, den0, num0, message, W_fc,
                     b_fc.reshape(1, HC), W_l, wb)
    return outp[:N]


# speedup vs baseline: 289.6241x; 1.5327x over previous
import functools

import jax
import jax.numpy as jnp
from jax import lax
from jax.experimental import pallas as pl
from jax.experimental.pallas import tpu as pltpu
from jax.experimental.pallas import tpu_sc as plsc

N = 50000
E = 800000
H = 2
C = 32
HC = H * C
HID = 128
B = 32

NC = 2
NS = 16
NW = NC * NS
L = 16

RPW = 208
EPW = RPW * 128
E_PAD = NW * EPW
E_ROWS = E_PAD // 128

N_PAD = 53248
N_ROWS = N_PAD // 128
SLC = N_PAD // NS

CH = 8
GCH = RPW // CH

DEAD = N

EBLK = 128
NBLK = 8
GRID = E_ROWS // EBLK

def _mesh():
    return plsc.VectorSubcoreMesh(
        core_axis_name="c", subcore_axis_name="s",
        num_cores=NC, num_subcores=NS)


@functools.cache
def _build_sc_pass1():
    return functools.partial(
        pl.kernel,
        out_type=(
            jax.ShapeDtypeStruct((NC * N_PAD,), jnp.float32),
            jax.ShapeDtypeStruct((NC * N_PAD,), jnp.float32),
            jax.ShapeDtypeStruct((E_ROWS, 128), jnp.float32),
            jax.ShapeDtypeStruct((E_ROWS, 128), jnp.float32),
        ),
        mesh=_mesh(),
        compiler_params=pltpu.CompilerParams(needs_layout_passes=False),
        scratch_types=[
            pltpu.VMEM((N_PAD,), jnp.float32),
            pltpu.VMEM((2 * CH, 128), jnp.int32),
            pltpu.VMEM((2 * CH, 128), jnp.int32),
            pltpu.VMEM((2 * CH, 128), jnp.float32),
            pltpu.VMEM((2 * CH, 128), jnp.float32),
            pltpu.VMEM((2 * CH, 128), jnp.float32),
            pltpu.VMEM((128,), jnp.float32),
            pltpu.VMEM((SLC,), jnp.float32),
            pltpu.VMEM_SHARED((N_PAD,), jnp.float32),
            pltpu.VMEM_SHARED((N_PAD,), jnp.float32),
            pltpu.SemaphoreType.DMA,
            pltpu.SemaphoreType.DMA,
            pltpu.SemaphoreType.DMA,
            pltpu.SemaphoreType.DMA,
        ],
    )(_sc_pass1_body)


def _zero_vmem(buf, n):
    def zb(i, carry):
        buf[pl.ds(i * L, L)] = jnp.zeros((L,), jnp.float32)
        return carry
    lax.fori_loop(0, n // L, zb, 0)


def _sc_pass1_body(x_hbm, src_hbm, dst_hbm, ea_hbm,
                   cnt_out, sum_out, xs_out, xd_out,
                   x_v, srcb, dstb, eab, xsb, xdb, onesb, zbuf,
                   cnt_acc, sum_acc, semfa, semfb, semia, semib):
    cid = lax.axis_index("c")
    sid = lax.axis_index("s")
    wid = cid * NS + sid
    for k in range(8):
        onesb[pl.ds(k * L, L)] = jnp.ones((L,), jnp.float32)
    _zero_vmem(zbuf, SLC)
    pltpu.sync_copy(x_hbm, x_v)
    off = sid * SLC
    pltpu.sync_copy(zbuf, cnt_acc.at[pl.ds(off, SLC)])
    pltpu.sync_copy(zbuf, sum_acc.at[pl.ds(off, SLC)])
    plsc.subcore_barrier()

    rbase = wid * RPW

    def body(g, carry):
        rb = rbase + g * CH
        rows = pl.ds(0, CH)
        pltpu.sync_copy(src_hbm.at[pl.ds(rb, CH)], srcb.at[rows])
        pltpu.sync_copy(dst_hbm.at[pl.ds(rb, CH)], dstb.at[rows])
        pltpu.sync_copy(ea_hbm.at[pl.ds(rb, CH)], eab.at[rows])
        for j in range(CH):
            for k in range(8):
                sl = pl.ds(k * L, L)
                xsb[j, sl] = plsc.load_gather(x_v, [srcb[j, sl]])
                xdb[j, sl] = plsc.load_gather(x_v, [dstb[j, sl]])
        descs = [
            pltpu.async_copy(xsb.at[rows], xs_out.at[pl.ds(rb, CH)], semia),
            pltpu.async_copy(xdb.at[rows], xd_out.at[pl.ds(rb, CH)], semia),
        ]
        for j in range(CH):
            descs.append(pltpu.async_copy(
                onesb, cnt_acc.at[dstb.at[j]], semib, add=True))
            descs.append(pltpu.async_copy(
                eab.at[j], sum_acc.at[dstb.at[j]], semib, add=True))
        for d in descs:
            d.wait()
        return carry

    lax.fori_loop(0, GCH, body, 0)
    plsc.subcore_barrier()
    pltpu.sync_copy(cnt_acc.at[pl.ds(off, SLC)], zbuf)
    pltpu.sync_copy(zbuf, cnt_out.at[pl.ds(cid * N_PAD + off, SLC)])
    pltpu.sync_copy(sum_acc.at[pl.ds(off, SLC)], zbuf)
    pltpu.sync_copy(zbuf, sum_out.at[pl.ds(cid * N_PAD + off, SLC)])


@functools.cache
def _build_sc_pass2():
    return functools.partial(
        pl.kernel,
        out_type=tuple(jax.ShapeDtypeStruct((NC * N_PAD,), jnp.float32)
                       for _ in range(4)),
        mesh=_mesh(),
        compiler_params=pltpu.CompilerParams(needs_layout_passes=False),
        scratch_types=[
            pltpu.VMEM((2 * CH, 128), jnp.int32),
            pltpu.VMEM((2 * CH, 128), jnp.float32),
            pltpu.VMEM((2 * CH, 128), jnp.float32),
            pltpu.VMEM((2 * CH, 128), jnp.float32),
            pltpu.VMEM((2 * CH, 128), jnp.float32),
            pltpu.VMEM((SLC,), jnp.float32),
            pltpu.VMEM_SHARED((N_PAD,), jnp.float32),
            pltpu.VMEM_SHARED((N_PAD,), jnp.float32),
            pltpu.VMEM_SHARED((N_PAD,), jnp.float32),
            pltpu.VMEM_SHARED((N_PAD,), jnp.float32),
            pltpu.SemaphoreType.DMA,
            pltpu.SemaphoreType.DMA,
            pltpu.SemaphoreType.DMA,
            pltpu.SemaphoreType.DMA,
        ],
    )(_sc_pass2_body)


def _sc_pass2_body(dst_hbm, p0_hbm, p1_hbm, q0_hbm, q1_hbm,
                   o0, o1, o2, o3,
                   dstb, p0b, p1b, q0b, q1b, zbuf, a0, a1, a2, a3,
                   semfa, semfb, semia, semib):
    cid = lax.axis_index("c")
    sid = lax.axis_index("s")
    wid = cid * NS + sid
    off = sid * SLC
    _zero_vmem(zbuf, SLC)
    for a in (a0, a1, a2, a3):
        pltpu.sync_copy(zbuf, a.at[pl.ds(off, SLC)])
    plsc.subcore_barrier()

    rbase = wid * RPW
    ins = (dst_hbm, p0_hbm, p1_hbm, q0_hbm, q1_hbm)
    bufs = (dstb, p0b, p1b, q0b, q1b)

    def body(g, carry):
        rb = rbase + g * CH
        rows = pl.ds(0, CH)
        fdescs = [pltpu.async_copy(h.at[pl.ds(rb, CH)], b.at[rows], semfa)
                  for h, b in zip(ins, bufs)]
        for d in fdescs:
            d.wait()
        descs = []
        for j in range(CH):
            idx = dstb.at[j]
            for b, a in zip((p0b, p1b, q0b, q1b), (a0, a1, a2, a3)):
                descs.append(pltpu.async_copy(
                    b.at[j], a.at[idx], semib, add=True))
        for d in descs:
            d.wait()
        return carry

    lax.fori_loop(0, GCH, body, 0)
    plsc.subcore_barrier()
    for o, a in zip((o0, o1, o2, o3), (a0, a1, a2, a3)):
        pltpu.sync_copy(a.at[pl.ds(off, SLC)], zbuf)
        pltpu.sync_copy(zbuf, o.at[pl.ds(cid * N_PAD + off, SLC)])


def _tc_edge_body(xs_r, xd_r, ea_r, x_r, cnt_r, sum_r,
                  wl_r, wr_r, we_r, bb_r, att_r,
                  p0_r, p1_r, q0_r, q1_r, den0_r, num0_r):
    xs = xs_r[...]
    xd = xd_r[...]
    ea = ea_r[...]
    l0 = jnp.zeros_like(xs)
    l1 = jnp.zeros_like(xs)
    for c in range(HC):
        m = xs * wl_r[0, c] + xd * wr_r[0, c] + ea * we_r[0, c] + bb_r[0, c]
        m = jnp.maximum(m, 0.2 * m)
        if c < C:
            l0 = l0 + att_r[0, c] * m
        else:
            l1 = l1 + att_r[0, c] * m
    p0 = jnp.exp(l0)
    p1 = jnp.exp(l1)
    p0_r[...] = p0
    p1_r[...] = p1
    q0_r[...] = p0 * xs
    q1_r[...] = p1 * xs

    xn = x_r[...]
    cnt = cnt_r[0] + cnt_r[1]
    sume = sum_r[0] + sum_r[1]
    la = sume / jnp.maximum(cnt, 1.0)
    s0 = jnp.zeros_like(xn)
    s1 = jnp.zeros_like(xn)
    for c in range(HC):
        m = xn * (wl_r[0, c] + wr_r[0, c]) + la * we_r[0, c] + bb_r[0, c]
        m = jnp.maximum(m, 0.2 * m)
        if c < C:
            s0 = s0 + att_r[0, c] * m
        else:
            s1 = s1 + att_r[0, c] * m
    e0 = jnp.exp(s0)
    e1 = jnp.exp(s1)
    den0_r[0] = e0
    den0_r[1] = e1
    num0_r[0] = e0 * xn
    num0_r[1] = e1 * xn


def _tc_pass(xs2d, xd2d, ea2d, x2d, cnt2d, sum2d, wl, wr, we, bb, att_f):
    espec = pl.BlockSpec((EBLK, 128), lambda i: (i, 0))
    nspec = pl.BlockSpec((NBLK, 128), lambda i: (i, 0))
    hspec = pl.BlockSpec((2, NBLK, 128), lambda i: (0, i, 0))
    sspec = pl.BlockSpec(memory_space=pltpu.SMEM)
    return pl.pallas_call(
        _tc_edge_body,
        grid=(GRID,),
        in_specs=[espec, espec, espec, nspec, hspec, hspec,
                  sspec, sspec, sspec, sspec, sspec],
        out_specs=[espec, espec, espec, espec, hspec, hspec],
        out_shape=[
            jax.ShapeDtypeStruct((E_ROWS, 128), jnp.float32),
            jax.ShapeDtypeStruct((E_ROWS, 128), jnp.float32),
            jax.ShapeDtypeStruct((E_ROWS, 128), jnp.float32),
            jax.ShapeDtypeStruct((E_ROWS, 128), jnp.float32),
            jax.ShapeDtypeStruct((2, N_ROWS, 128), jnp.float32),
            jax.ShapeDtypeStruct((2, N_ROWS, 128), jnp.float32),
        ],
    )(xs2d, xd2d, ea2d, x2d, cnt2d, sum2d, wl, wr, we, bb, att_f)


def _tc_final_body(c0_r, c1_r, c2_r, c3_r, den0_r, num0_r,
                   msg_r, wfc_r, bfc_r, wl_r, wb_r, out_r):
    me = lax.dot_general(msg_r[...], wfc_r[...],
                         (((1,), (1,)), ((), ())),
                         preferred_element_type=jnp.float32)
    me = me + bfc_r[...]
    mw = me * wl_r[...]
    u0 = jnp.sum(mw[:, :C], axis=1, keepdims=True)
    u1 = jnp.sum(mw[:, C:], axis=1, keepdims=True)
    w = jnp.sum(me * wb_r[...], axis=1, keepdims=True)

    den0_v = c0_r[0] + c0_r[1] + den0_r[0]
    den1_v = c1_r[0] + c1_r[1] + den0_r[1]
    num0_v = c2_r[0] + c2_r[1] + num0_r[0]
    num1_v = c3_r[0] + c3_r[1] + num0_r[1]
    sx0 = num0_v / den0_v
    sx1 = num1_v / den1_v
    for g in range(NBLK):
        d = u0 * sx0[g:g + 1, :] + u1 * sx1[g:g + 1, :] + w
        mx = jnp.max(d, axis=0, keepdims=True)
        ex = jnp.exp(d - mx)
        sm = jnp.sum(ex, axis=0, keepdims=True)
        out_r[pl.ds(g * 128, 128), :] = (ex / sm).T


def _tc_final(c0, c1, c2, c3, den0, num0, msg, wfc, bfc, wl, wb):
    hspec = pl.BlockSpec((2, NBLK, 128), lambda i: (0, i, 0))
    wspec = lambda shape: pl.BlockSpec(shape, lambda i: tuple(0 for _ in shape))
    return pl.pallas_call(
        _tc_final_body,
        grid=(GRID,),
        in_specs=[hspec, hspec, hspec, hspec, hspec, hspec,
                  wspec((B, HID)), wspec((HC, HID)), wspec((1, HC)),
                  wspec((1, HC)), wspec((1, HC))],
        out_specs=pl.BlockSpec((NBLK * 128, B), lambda i: (i, 0)),
        out_shape=jax.ShapeDtypeStruct((N_PAD, B), jnp.float32),
    )(c0, c1, c2, c3, den0, num0, msg, wfc, bfc, wl, wb)


def kernel(x, edge_index, edge_attr, message,
           W_l, b_l, W_r, b_r, W_e, att, bias, W_fc, b_fc):
    src = edge_index[0]
    dst = edge_index[1]
    ea = edge_attr[:, 0]
    pad = E_PAD - E
    pad_i = jnp.arange(pad, dtype=jnp.int32)
    src2d = jnp.concatenate([src, pad_i % N]).reshape(E_ROWS, 128)
    dst2d = jnp.concatenate(
        [dst, DEAD + pad_i % (N_PAD - N)]).reshape(E_ROWS, 128)
    ea2d = jnp.pad(ea, (0, pad)).reshape(E_ROWS, 128)
    x_flat = jnp.pad(x[:, 0], (0, N_PAD - N))
    x2d = x_flat.reshape(N_ROWS, 128)

    cnt1, sum1, xs2d, xd2d = _build_sc_pass1()(x_flat, src2d, dst2d, ea2d)
    cnt2d = cnt1.reshape(NC, N_ROWS, 128)
    sum2d = sum1.reshape(NC, N_ROWS, 128)

    bb = (b_l + b_r).reshape(1, HC)
    att_f = att.reshape(1, HC)
    p0, p1, q0, q1, den0, num0 = _tc_pass(
        xs2d, xd2d, ea2d, x2d, cnt2d, sum2d, W_l, W_r, W_e, bb, att_f)

    accs = _build_sc_pass2()(dst2d, p0, p1, q0, q1)
    c0, c1, c2, c3 = (a.reshape(NC, N_ROWS, 128) for a in accs)

    wb = (b_l + bias).reshape(1, HC)
    outp = _tc_final(c0, c1, c2, c3, den0, num0, message, W_fc,
                     b_fc.reshape(1, HC), W_l, wb)
    return outp[:N]


# speedup vs baseline: 299.6208x; 1.0345x over previous
import functools

import jax
import jax.numpy as jnp
from jax import lax
from jax.experimental import pallas as pl
from jax.experimental.pallas import tpu as pltpu
from jax.experimental.pallas import tpu_sc as plsc

N = 50000
E = 800000
H = 2
C = 32
HC = H * C
HID = 128
B = 32

NC = 2
NS = 16
NW = NC * NS
L = 16

RPW = 208
EPW = RPW * 128
E_PAD = NW * EPW
E_ROWS = E_PAD // 128

N_PAD = 53248
N_ROWS = N_PAD // 128
SLC = N_PAD // NS

CH = 8
GCH = RPW // CH

DEAD = N

EBLK = 128
NBLK = 8
GRID = E_ROWS // EBLK

def _mesh():
    return plsc.VectorSubcoreMesh(
        core_axis_name="c", subcore_axis_name="s",
        num_cores=NC, num_subcores=NS)


@functools.cache
def _build_sc_pass1():
    return functools.partial(
        pl.kernel,
        out_type=(
            jax.ShapeDtypeStruct((NC * N_PAD,), jnp.float32),
            jax.ShapeDtypeStruct((NC * N_PAD,), jnp.float32),
            jax.ShapeDtypeStruct((E_ROWS, 128), jnp.float32),
            jax.ShapeDtypeStruct((E_ROWS, 128), jnp.float32),
        ),
        mesh=_mesh(),
        compiler_params=pltpu.CompilerParams(needs_layout_passes=False),
        scratch_types=[
            pltpu.VMEM((N_PAD,), jnp.float32),
            pltpu.VMEM((2 * CH, 128), jnp.int32),
            pltpu.VMEM((2 * CH, 128), jnp.int32),
            pltpu.VMEM((2 * CH, 128), jnp.float32),
            pltpu.VMEM((2 * CH, 128), jnp.float32),
            pltpu.VMEM((2 * CH, 128), jnp.float32),
            pltpu.VMEM((128,), jnp.float32),
            pltpu.VMEM((SLC,), jnp.float32),
            pltpu.VMEM_SHARED((N_PAD,), jnp.float32),
            pltpu.VMEM_SHARED((N_PAD,), jnp.float32),
            pltpu.SemaphoreType.DMA,
            pltpu.SemaphoreType.DMA,
            pltpu.SemaphoreType.DMA,
            pltpu.SemaphoreType.DMA,
        ],
    )(_sc_pass1_body)


def _zero_vmem(buf, n):
    def zb(i, carry):
        buf[pl.ds(i * L, L)] = jnp.zeros((L,), jnp.float32)
        return carry
    lax.fori_loop(0, n // L, zb, 0)


def _sc_pass1_body(x_hbm, src_hbm, dst_hbm, ea_hbm,
                   cnt_out, sum_out, xs_out, xd_out,
                   x_v, srcb, dstb, eab, xsb, xdb, onesb, zbuf,
                   cnt_acc, sum_acc, semfa, semfb, semia, semib):
    cid = lax.axis_index("c")
    sid = lax.axis_index("s")
    wid = cid * NS + sid
    for k in range(8):
        onesb[pl.ds(k * L, L)] = jnp.ones((L,), jnp.float32)
    _zero_vmem(zbuf, SLC)
    pltpu.sync_copy(x_hbm, x_v)
    off = sid * SLC
    pltpu.sync_copy(zbuf, cnt_acc.at[pl.ds(off, SLC)])
    pltpu.sync_copy(zbuf, sum_acc.at[pl.ds(off, SLC)])
    plsc.subcore_barrier()

    rbase = wid * RPW

    def gathers(s):
        for j in range(CH):
            row = s * CH + j
            for k in range(8):
                sl = pl.ds(k * L, L)
                xsb[row, sl] = plsc.load_gather(x_v, [srcb[row, sl]])
                xdb[row, sl] = plsc.load_gather(x_v, [dstb[row, sl]])

    def issue(rb, s, semo, sems):
        rows = pl.ds(s * CH, CH)
        outs = [
            pltpu.async_copy(xsb.at[rows], xs_out.at[pl.ds(rb, CH)], semo),
            pltpu.async_copy(xdb.at[rows], xd_out.at[pl.ds(rb, CH)], semo),
        ]
        scats = []
        for j in range(CH):
            row = s * CH + j
            scats.append(pltpu.async_copy(
                onesb, cnt_acc.at[dstb.at[row]], sems, add=True))
            scats.append(pltpu.async_copy(
                eab.at[row], sum_acc.at[dstb.at[row]], sems, add=True))
        return outs, scats

    def body(i, carry):
        rb0 = rbase + (2 * i) * CH
        rb1 = rb0 + CH
        rowsB = pl.ds(CH, CH)
        fb = [pltpu.async_copy(src_hbm.at[pl.ds(rb1, CH)], srcb.at[rowsB],
                               semfb),
              pltpu.async_copy(dst_hbm.at[pl.ds(rb1, CH)], dstb.at[rowsB],
                               semfb),
              pltpu.async_copy(ea_hbm.at[pl.ds(rb1, CH)], eab.at[rowsB],
                               semfb)]
        rowsA = pl.ds(0, CH)
        pltpu.sync_copy(src_hbm.at[pl.ds(rb0, CH)], srcb.at[rowsA])
        pltpu.sync_copy(dst_hbm.at[pl.ds(rb0, CH)], dstb.at[rowsA])
        pltpu.sync_copy(ea_hbm.at[pl.ds(rb0, CH)], eab.at[rowsA])
        gathers(0)
        outsa, scatsa = issue(rb0, 0, semfa, semia)
        for d in fb:
            d.wait()
        gathers(1)
        outsb, scatsb = issue(rb1, 1, semfa, semib)
        for d in scatsa + outsa + scatsb + outsb:
            d.wait()
        return carry

    lax.fori_loop(0, GCH // 2, body, 0)
    plsc.subcore_barrier()
    pltpu.sync_copy(cnt_acc.at[pl.ds(off, SLC)], zbuf)
    pltpu.sync_copy(zbuf, cnt_out.at[pl.ds(cid * N_PAD + off, SLC)])
    pltpu.sync_copy(sum_acc.at[pl.ds(off, SLC)], zbuf)
    pltpu.sync_copy(zbuf, sum_out.at[pl.ds(cid * N_PAD + off, SLC)])


@functools.cache
def _build_sc_pass2():
    return functools.partial(
        pl.kernel,
        out_type=tuple(jax.ShapeDtypeStruct((NC * N_PAD,), jnp.float32)
                       for _ in range(4)),
        mesh=_mesh(),
        compiler_params=pltpu.CompilerParams(needs_layout_passes=False),
        scratch_types=[
            pltpu.VMEM((2 * CH, 128), jnp.int32),
            pltpu.VMEM((2 * CH, 128), jnp.float32),
            pltpu.VMEM((2 * CH, 128), jnp.float32),
            pltpu.VMEM((2 * CH, 128), jnp.float32),
            pltpu.VMEM((2 * CH, 128), jnp.float32),
            pltpu.VMEM((SLC,), jnp.float32),
            pltpu.VMEM_SHARED((N_PAD,), jnp.float32),
            pltpu.VMEM_SHARED((N_PAD,), jnp.float32),
            pltpu.VMEM_SHARED((N_PAD,), jnp.float32),
            pltpu.VMEM_SHARED((N_PAD,), jnp.float32),
            pltpu.SemaphoreType.DMA,
            pltpu.SemaphoreType.DMA,
            pltpu.SemaphoreType.DMA,
            pltpu.SemaphoreType.DMA,
        ],
    )(_sc_pass2_body)


def _sc_pass2_body(dst_hbm, p0_hbm, p1_hbm, q0_hbm, q1_hbm,
                   o0, o1, o2, o3,
                   dstb, p0b, p1b, q0b, q1b, zbuf, a0, a1, a2, a3,
                   semfa, semfb, semia, semib):
    cid = lax.axis_index("c")
    sid = lax.axis_index("s")
    wid = cid * NS + sid
    off = sid * SLC
    _zero_vmem(zbuf, SLC)
    for a in (a0, a1, a2, a3):
        pltpu.sync_copy(zbuf, a.at[pl.ds(off, SLC)])
    plsc.subcore_barrier()

    rbase = wid * RPW
    ins = (dst_hbm, p0_hbm, p1_hbm, q0_hbm, q1_hbm)
    bufs = (dstb, p0b, p1b, q0b, q1b)

    def issue(s, sem):
        descs = []
        for j in range(CH):
            row = s * CH + j
            idx = dstb.at[row]
            for b, a in zip((p0b, p1b, q0b, q1b), (a0, a1, a2, a3)):
                descs.append(pltpu.async_copy(
                    b.at[row], a.at[idx], sem, add=True))
        return descs

    def body(i, carry):
        rb0 = rbase + (2 * i) * CH
        rb1 = rb0 + CH
        rowsB = pl.ds(CH, CH)
        fb = [pltpu.async_copy(h.at[pl.ds(rb1, CH)], b.at[rowsB], semfb)
              for h, b in zip(ins, bufs)]
        rowsA = pl.ds(0, CH)
        for h, b in zip(ins, bufs):
            pltpu.sync_copy(h.at[pl.ds(rb0, CH)], b.at[rowsA])
        da = issue(0, semia)
        for d in fb:
            d.wait()
        for d in da:
            d.wait()
        db = issue(1, semib)
        for d in db:
            d.wait()
        return carry

    lax.fori_loop(0, GCH // 2, body, 0)
    plsc.subcore_barrier()
    for o, a in zip((o0, o1, o2, o3), (a0, a1, a2, a3)):
        pltpu.sync_copy(a.at[pl.ds(off, SLC)], zbuf)
        pltpu.sync_copy(zbuf, o.at[pl.ds(cid * N_PAD + off, SLC)])


def _tc_edge_body(xs_r, xd_r, ea_r, x_r, cnt_r, sum_r,
                  wl_r, wr_r, we_r, bb_r, att_r,
                  p0_r, p1_r, q0_r, q1_r, den0_r, num0_r):
    xs = xs_r[...]
    xd = xd_r[...]
    ea = ea_r[...]
    l0 = jnp.zeros_like(xs)
    l1 = jnp.zeros_like(xs)
    for c in range(HC):
        m = xs * wl_r[0, c] + xd * wr_r[0, c] + ea * we_r[0, c] + bb_r[0, c]
        m = jnp.maximum(m, 0.2 * m)
        if c < C:
            l0 = l0 + att_r[0, c] * m
        else:
            l1 = l1 + att_r[0, c] * m
    p0 = jnp.exp(l0)
    p1 = jnp.exp(l1)
    p0_r[...] = p0
    p1_r[...] = p1
    q0_r[...] = p0 * xs
    q1_r[...] = p1 * xs

    xn = x_r[...]
    cnt = cnt_r[0] + cnt_r[1]
    sume = sum_r[0] + sum_r[1]
    la = sume / jnp.maximum(cnt, 1.0)
    s0 = jnp.zeros_like(xn)
    s1 = jnp.zeros_like(xn)
    for c in range(HC):
        m = xn * (wl_r[0, c] + wr_r[0, c]) + la * we_r[0, c] + bb_r[0, c]
        m = jnp.maximum(m, 0.2 * m)
        if c < C:
            s0 = s0 + att_r[0, c] * m
        else:
            s1 = s1 + att_r[0, c] * m
    e0 = jnp.exp(s0)
    e1 = jnp.exp(s1)
    den0_r[0] = e0
    den0_r[1] = e1
    num0_r[0] = e0 * xn
    num0_r[1] = e1 * xn


def _tc_pass(xs2d, xd2d, ea2d, x2d, cnt2d, sum2d, wl, wr, we, bb, att_f):
    espec = pl.BlockSpec((EBLK, 128), lambda i: (i, 0))
    nspec = pl.BlockSpec((NBLK, 128), lambda i: (i, 0))
    hspec = pl.BlockSpec((2, NBLK, 128), lambda i: (0, i, 0))
    sspec = pl.BlockSpec(memory_space=pltpu.SMEM)
    return pl.pallas_call(
        _tc_edge_body,
        grid=(GRID,),
        in_specs=[espec, espec, espec, nspec, hspec, hspec,
                  sspec, sspec, sspec, sspec, sspec],
        out_specs=[espec, espec, espec, espec, hspec, hspec],
        out_shape=[
            jax.ShapeDtypeStruct((E_ROWS, 128), jnp.float32),
            jax.ShapeDtypeStruct((E_ROWS, 128), jnp.float32),
            jax.ShapeDtypeStruct((E_ROWS, 128), jnp.float32),
            jax.ShapeDtypeStruct((E_ROWS, 128), jnp.float32),
            jax.ShapeDtypeStruct((2, N_ROWS, 128), jnp.float32),
            jax.ShapeDtypeStruct((2, N_ROWS, 128), jnp.float32),
        ],
    )(xs2d, xd2d, ea2d, x2d, cnt2d, sum2d, wl, wr, we, bb, att_f)


def _tc_final_body(c0_r, c1_r, c2_r, c3_r, den0_r, num0_r,
                   msg_r, wfc_r, bfc_r, wl_r, wb_r, out_r):
    me = lax.dot_general(msg_r[...], wfc_r[...],
                         (((1,), (1,)), ((), ())),
                         preferred_element_type=jnp.float32)
    me = me + bfc_r[...]
    mw = me * wl_r[...]
    u0 = jnp.sum(mw[:, :C], axis=1, keepdims=True)
    u1 = jnp.sum(mw[:, C:], axis=1, keepdims=True)
    w = jnp.sum(me * wb_r[...], axis=1, keepdims=True)

    den0_v = c0_r[0] + c0_r[1] + den0_r[0]
    den1_v = c1_r[0] + c1_r[1] + den0_r[1]
    num0_v = c2_r[0] + c2_r[1] + num0_r[0]
    num1_v = c3_r[0] + c3_r[1] + num0_r[1]
    sx0 = num0_v / den0_v
    sx1 = num1_v / den1_v
    for g in range(NBLK):
        d = u0 * sx0[g:g + 1, :] + u1 * sx1[g:g + 1, :] + w
        mx = jnp.max(d, axis=0, keepdims=True)
        ex = jnp.exp(d - mx)
        sm = jnp.sum(ex, axis=0, keepdims=True)
        out_r[pl.ds(g * 128, 128), :] = (ex / sm).T


def _tc_final(c0, c1, c2, c3, den0, num0, msg, wfc, bfc, wl, wb):
    hspec = pl.BlockSpec((2, NBLK, 128), lambda i: (0, i, 0))
    wspec = lambda shape: pl.BlockSpec(shape, lambda i: tuple(0 for _ in shape))
    return pl.pallas_call(
        _tc_final_body,
        grid=(GRID,),
        in_specs=[hspec, hspec, hspec, hspec, hspec, hspec,
                  wspec((B, HID)), wspec((HC, HID)), wspec((1, HC)),
                  wspec((1, HC)), wspec((1, HC))],
        out_specs=pl.BlockSpec((NBLK * 128, B), lambda i: (i, 0)),
        out_shape=jax.ShapeDtypeStruct((N_PAD, B), jnp.float32),
    )(c0, c1, c2, c3, den0, num0, msg, wfc, bfc, wl, wb)


def kernel(x, edge_index, edge_attr, message,
           W_l, b_l, W_r, b_r, W_e, att, bias, W_fc, b_fc):
    src = edge_index[0]
    dst = edge_index[1]
    ea = edge_attr[:, 0]
    pad = E_PAD - E
    pad_i = jnp.arange(pad, dtype=jnp.int32)
    src2d = jnp.concatenate([src, pad_i % N]).reshape(E_ROWS, 128)
    dst2d = jnp.concatenate(
        [dst, DEAD + pad_i % (N_PAD - N)]).reshape(E_ROWS, 128)
    ea2d = jnp.pad(ea, (0, pad)).reshape(E_ROWS, 128)
    x_flat = jnp.pad(x[:, 0], (0, N_PAD - N))
    x2d = x_flat.reshape(N_ROWS, 128)

    cnt1, sum1, xs2d, xd2d = _build_sc_pass1()(x_flat, src2d, dst2d, ea2d)
    cnt2d = cnt1.reshape(NC, N_ROWS, 128)
    sum2d = sum1.reshape(NC, N_ROWS, 128)

    bb = (b_l + b_r).reshape(1, HC)
    att_f = att.reshape(1, HC)
    p0, p1, q0, q1, den0, num0 = _tc_pass(
        xs2d, xd2d, ea2d, x2d, cnt2d, sum2d, W_l, W_r, W_e, bb, att_f)

    accs = _build_sc_pass2()(dst2d, p0, p1, q0, q1)
    c0, c1, c2, c3 = (a.reshape(NC, N_ROWS, 128) for a in accs)

    wb = (b_l + bias).reshape(1, HC)
    outp = _tc_final(c0, c1, c2, c3, den0, num0, message, W_fc,
                     b_fc.reshape(1, HC), W_l, wb)
    return outp[:N]


# speedup vs baseline: 333.6066x; 1.1134x over previous
import functools

import jax
import jax.numpy as jnp
from jax import lax
from jax.experimental import pallas as pl
from jax.experimental.pallas import tpu as pltpu
from jax.experimental.pallas import tpu_sc as plsc

N = 50000
E = 800000
H = 2
C = 32
HC = H * C
HID = 128
B = 32

NC = 2
NS = 16
NW = NC * NS
L = 16

RPW = 208
EPW = RPW * 128
E_PAD = NW * EPW
E_ROWS = E_PAD // 128

N_PAD = 53248
N_ROWS = N_PAD // 128
SLC = N_PAD // NS

CH = 8
GCH = RPW // CH

DEAD = N

EBLK = 128
NBLK = 8
GRID = E_ROWS // EBLK

def _mesh():
    return plsc.VectorSubcoreMesh(
        core_axis_name="c", subcore_axis_name="s",
        num_cores=NC, num_subcores=NS)


@functools.cache
def _build_sc_pass1():
    return functools.partial(
        pl.kernel,
        out_type=(
            jax.ShapeDtypeStruct((NC * N_PAD,), jnp.float32),
            jax.ShapeDtypeStruct((NC * N_PAD,), jnp.float32),
            jax.ShapeDtypeStruct((E_ROWS, 128), jnp.float32),
            jax.ShapeDtypeStruct((E_ROWS, 128), jnp.float32),
        ),
        mesh=_mesh(),
        compiler_params=pltpu.CompilerParams(needs_layout_passes=False),
        scratch_types=[
            pltpu.VMEM((N_PAD,), jnp.float32),
            pltpu.VMEM((2 * CH, 128), jnp.int32),
            pltpu.VMEM((2 * CH, 128), jnp.int32),
            pltpu.VMEM((2 * CH, 128), jnp.float32),
            pltpu.VMEM((2 * CH, 128), jnp.float32),
            pltpu.VMEM((2 * CH, 128), jnp.float32),
            pltpu.VMEM((128,), jnp.float32),
            pltpu.VMEM((SLC,), jnp.float32),
            pltpu.VMEM_SHARED((N_PAD,), jnp.float32),
            pltpu.VMEM_SHARED((N_PAD,), jnp.float32),
            pltpu.SemaphoreType.DMA,
            pltpu.SemaphoreType.DMA,
            pltpu.SemaphoreType.DMA,
            pltpu.SemaphoreType.DMA,
        ],
    )(_sc_pass1_body)


def _zero_vmem(buf, n):
    def zb(i, carry):
        buf[pl.ds(i * L, L)] = jnp.zeros((L,), jnp.float32)
        return carry
    lax.fori_loop(0, n // L, zb, 0)


def _sc_pass1_body(x_hbm, src_hbm, dst_hbm, ea_hbm,
                   cnt_out, sum_out, xs_out, xd_out,
                   x_v, srcb, dstb, eab, xsb, xdb, onesb, zbuf,
                   cnt_acc, sum_acc, semfa, semfb, semia, semib):
    cid = lax.axis_index("c")
    sid = lax.axis_index("s")
    wid = cid * NS + sid
    for k in range(8):
        onesb[pl.ds(k * L, L)] = jnp.ones((L,), jnp.float32)
    _zero_vmem(zbuf, SLC)
    pltpu.sync_copy(x_hbm, x_v)
    off = sid * SLC
    pltpu.sync_copy(zbuf, cnt_acc.at[pl.ds(off, SLC)])
    pltpu.sync_copy(zbuf, sum_acc.at[pl.ds(off, SLC)])
    plsc.subcore_barrier()

    rbase = wid * RPW

    def gathers(s):
        for j in range(CH):
            row = s * CH + j
            for k in range(8):
                sl = pl.ds(k * L, L)
                xsb[row, sl] = plsc.load_gather(x_v, [srcb[row, sl]])
                xdb[row, sl] = plsc.load_gather(x_v, [dstb[row, sl]])

    def issue(rb, s, semo, sems):
        rows = pl.ds(s * CH, CH)
        outs = [
            pltpu.async_copy(xsb.at[rows], xs_out.at[pl.ds(rb, CH)], semo),
            pltpu.async_copy(xdb.at[rows], xd_out.at[pl.ds(rb, CH)], semo),
        ]
        scats = []
        for j in range(CH):
            row = s * CH + j
            scats.append(pltpu.async_copy(
                onesb, cnt_acc.at[dstb.at[row]], sems, add=True))
            scats.append(pltpu.async_copy(
                eab.at[row], sum_acc.at[dstb.at[row]], sems, add=True))
        return outs, scats

    def body(i, carry):
        rb0 = rbase + (2 * i) * CH
        rb1 = rb0 + CH
        rowsB = pl.ds(CH, CH)
        fb = [pltpu.async_copy(src_hbm.at[pl.ds(rb1, CH)], srcb.at[rowsB],
                               semfb),
              pltpu.async_copy(dst_hbm.at[pl.ds(rb1, CH)], dstb.at[rowsB],
                               semfb),
              pltpu.async_copy(ea_hbm.at[pl.ds(rb1, CH)], eab.at[rowsB],
                               semfb)]
        rowsA = pl.ds(0, CH)
        fa = [pltpu.async_copy(src_hbm.at[pl.ds(rb0, CH)], srcb.at[rowsA],
                               semfa),
              pltpu.async_copy(dst_hbm.at[pl.ds(rb0, CH)], dstb.at[rowsA],
                               semfa),
              pltpu.async_copy(ea_hbm.at[pl.ds(rb0, CH)], eab.at[rowsA],
                               semfa)]
        for d in fa:
            d.wait()
        gathers(0)
        outsa, scatsa = issue(rb0, 0, semfa, semia)
        for d in fb:
            d.wait()
        gathers(1)
        outsb, scatsb = issue(rb1, 1, semfa, semib)
        for d in scatsa + outsa + scatsb + outsb:
            d.wait()
        return carry

    lax.fori_loop(0, GCH // 2, body, 0)
    plsc.subcore_barrier()
    pltpu.sync_copy(cnt_acc.at[pl.ds(off, SLC)], zbuf)
    pltpu.sync_copy(zbuf, cnt_out.at[pl.ds(cid * N_PAD + off, SLC)])
    pltpu.sync_copy(sum_acc.at[pl.ds(off, SLC)], zbuf)
    pltpu.sync_copy(zbuf, sum_out.at[pl.ds(cid * N_PAD + off, SLC)])


@functools.cache
def _build_sc_pass2():
    return functools.partial(
        pl.kernel,
        out_type=tuple(jax.ShapeDtypeStruct((NC * N_PAD,), jnp.float32)
                       for _ in range(4)),
        mesh=_mesh(),
        compiler_params=pltpu.CompilerParams(needs_layout_passes=False),
        scratch_types=[
            pltpu.VMEM((2 * CH, 128), jnp.int32),
            pltpu.VMEM((2 * CH, 128), jnp.float32),
            pltpu.VMEM((2 * CH, 128), jnp.float32),
            pltpu.VMEM((2 * CH, 128), jnp.float32),
            pltpu.VMEM((2 * CH, 128), jnp.float32),
            pltpu.VMEM((SLC,), jnp.float32),
            pltpu.VMEM_SHARED((N_PAD,), jnp.float32),
            pltpu.VMEM_SHARED((N_PAD,), jnp.float32),
            pltpu.VMEM_SHARED((N_PAD,), jnp.float32),
            pltpu.VMEM_SHARED((N_PAD,), jnp.float32),
            pltpu.SemaphoreType.DMA,
            pltpu.SemaphoreType.DMA,
            pltpu.SemaphoreType.DMA,
            pltpu.SemaphoreType.DMA,
        ],
    )(_sc_pass2_body)


def _sc_pass2_body(dst_hbm, p0_hbm, p1_hbm, q0_hbm, q1_hbm,
                   o0, o1, o2, o3,
                   dstb, p0b, p1b, q0b, q1b, zbuf, a0, a1, a2, a3,
                   semfa, semfb, semia, semib):
    cid = lax.axis_index("c")
    sid = lax.axis_index("s")
    wid = cid * NS + sid
    off = sid * SLC
    _zero_vmem(zbuf, SLC)
    for a in (a0, a1, a2, a3):
        pltpu.sync_copy(zbuf, a.at[pl.ds(off, SLC)])
    plsc.subcore_barrier()

    rbase = wid * RPW
    ins = (dst_hbm, p0_hbm, p1_hbm, q0_hbm, q1_hbm)
    bufs = (dstb, p0b, p1b, q0b, q1b)

    def issue(s, sem):
        descs = []
        for j in range(CH):
            row = s * CH + j
            idx = dstb.at[row]
            for b, a in zip((p0b, p1b, q0b, q1b), (a0, a1, a2, a3)):
                descs.append(pltpu.async_copy(
                    b.at[row], a.at[idx], sem, add=True))
        return descs

    def body(i, carry):
        rb0 = rbase + (2 * i) * CH
        rb1 = rb0 + CH
        rowsB = pl.ds(CH, CH)
        fb = [pltpu.async_copy(h.at[pl.ds(rb1, CH)], b.at[rowsB], semfb)
              for h, b in zip(ins, bufs)]
        rowsA = pl.ds(0, CH)
        fa = [pltpu.async_copy(h.at[pl.ds(rb0, CH)], b.at[rowsA], semfa)
              for h, b in zip(ins, bufs)]
        for d in fa:
            d.wait()
        da = issue(0, semia)
        for d in fb:
            d.wait()
        for d in da:
            d.wait()
        db = issue(1, semib)
        for d in db:
            d.wait()
        return carry

    lax.fori_loop(0, GCH // 2, body, 0)
    plsc.subcore_barrier()
    for o, a in zip((o0, o1, o2, o3), (a0, a1, a2, a3)):
        pltpu.sync_copy(a.at[pl.ds(off, SLC)], zbuf)
        pltpu.sync_copy(zbuf, o.at[pl.ds(cid * N_PAD + off, SLC)])


def _tc_edge_body(xs_r, xd_r, ea_r, x_r, cnt_r, sum_r,
                  wl_r, wr_r, we_r, bb_r, att_r,
                  p0_r, p1_r, q0_r, q1_r, den0_r, num0_r):
    xs = xs_r[...]
    xd = xd_r[...]
    ea = ea_r[...]
    l0 = jnp.zeros_like(xs)
    l1 = jnp.zeros_like(xs)
    for c in range(HC):
        m = xs * wl_r[0, c] + xd * wr_r[0, c] + ea * we_r[0, c] + bb_r[0, c]
        m = jnp.maximum(m, 0.2 * m)
        if c < C:
            l0 = l0 + att_r[0, c] * m
        else:
            l1 = l1 + att_r[0, c] * m
    p0 = jnp.exp(l0)
    p1 = jnp.exp(l1)
    p0_r[...] = p0
    p1_r[...] = p1
    q0_r[...] = p0 * xs
    q1_r[...] = p1 * xs

    xn = x_r[...]
    cnt = cnt_r[0] + cnt_r[1]
    sume = sum_r[0] + sum_r[1]
    la = sume / jnp.maximum(cnt, 1.0)
    s0 = jnp.zeros_like(xn)
    s1 = jnp.zeros_like(xn)
    for c in range(HC):
        m = xn * (wl_r[0, c] + wr_r[0, c]) + la * we_r[0, c] + bb_r[0, c]
        m = jnp.maximum(m, 0.2 * m)
        if c < C:
            s0 = s0 + att_r[0, c] * m
        else:
            s1 = s1 + att_r[0, c] * m
    e0 = jnp.exp(s0)
    e1 = jnp.exp(s1)
    den0_r[0] = e0
    den0_r[1] = e1
    num0_r[0] = e0 * xn
    num0_r[1] = e1 * xn


def _tc_pass(xs2d, xd2d, ea2d, x2d, cnt2d, sum2d, wl, wr, we, bb, att_f):
    espec = pl.BlockSpec((EBLK, 128), lambda i: (i, 0))
    nspec = pl.BlockSpec((NBLK, 128), lambda i: (i, 0))
    hspec = pl.BlockSpec((2, NBLK, 128), lambda i: (0, i, 0))
    sspec = pl.BlockSpec(memory_space=pltpu.SMEM)
    return pl.pallas_call(
        _tc_edge_body,
        grid=(GRID,),
        in_specs=[espec, espec, espec, nspec, hspec, hspec,
                  sspec, sspec, sspec, sspec, sspec],
        out_specs=[espec, espec, espec, espec, hspec, hspec],
        out_shape=[
            jax.ShapeDtypeStruct((E_ROWS, 128), jnp.float32),
            jax.ShapeDtypeStruct((E_ROWS, 128), jnp.float32),
            jax.ShapeDtypeStruct((E_ROWS, 128), jnp.float32),
            jax.ShapeDtypeStruct((E_ROWS, 128), jnp.float32),
            jax.ShapeDtypeStruct((2, N_ROWS, 128), jnp.float32),
            jax.ShapeDtypeStruct((2, N_ROWS, 128), jnp.float32),
        ],
    )(xs2d, xd2d, ea2d, x2d, cnt2d, sum2d, wl, wr, we, bb, att_f)


def _tc_final_body(c0_r, c1_r, c2_r, c3_r, den0_r, num0_r,
                   msg_r, wfc_r, bfc_r, wl_r, wb_r, out_r):
    me = lax.dot_general(msg_r[...], wfc_r[...],
                         (((1,), (1,)), ((), ())),
                         preferred_element_type=jnp.float32)
    me = me + bfc_r[...]
    mw = me * wl_r[...]
    u0 = jnp.sum(mw[:, :C], axis=1, keepdims=True)
    u1 = jnp.sum(mw[:, C:], axis=1, keepdims=True)
    w = jnp.sum(me * wb_r[...], axis=1, keepdims=True)

    den0_v = c0_r[0] + c0_r[1] + den0_r[0]
    den1_v = c1_r[0] + c1_r[1] + den0_r[1]
    num0_v = c2_r[0] + c2_r[1] + num0_r[0]
    num1_v = c3_r[0] + c3_r[1] + num0_r[1]
    sx0 = num0_v / den0_v
    sx1 = num1_v / den1_v
    for g in range(NBLK):
        d = u0 * sx0[g:g + 1, :] + u1 * sx1[g:g + 1, :] + w
        mx = jnp.max(d, axis=0, keepdims=True)
        ex = jnp.exp(d - mx)
        sm = jnp.sum(ex, axis=0, keepdims=True)
        out_r[pl.ds(g * 128, 128), :] = (ex / sm).T


def _tc_final(c0, c1, c2, c3, den0, num0, msg, wfc, bfc, wl, wb):
    hspec = pl.BlockSpec((2, NBLK, 128), lambda i: (0, i, 0))
    wspec = lambda shape: pl.BlockSpec(shape, lambda i: tuple(0 for _ in shape))
    return pl.pallas_call(
        _tc_final_body,
        grid=(GRID,),
        in_specs=[hspec, hspec, hspec, hspec, hspec, hspec,
                  wspec((B, HID)), wspec((HC, HID)), wspec((1, HC)),
                  wspec((1, HC)), wspec((1, HC))],
        out_specs=pl.BlockSpec((NBLK * 128, B), lambda i: (i, 0)),
        out_shape=jax.ShapeDtypeStruct((N_PAD, B), jnp.float32),
    )(c0, c1, c2, c3, den0, num0, msg, wfc, bfc, wl, wb)


def kernel(x, edge_index, edge_attr, message,
           W_l, b_l, W_r, b_r, W_e, att, bias, W_fc, b_fc):
    src = edge_index[0]
    dst = edge_index[1]
    ea = edge_attr[:, 0]
    pad = E_PAD - E
    pad_i = jnp.arange(pad, dtype=jnp.int32)
    src2d = jnp.concatenate([src, pad_i % N]).reshape(E_ROWS, 128)
    dst2d = jnp.concatenate(
        [dst, DEAD + pad_i % (N_PAD - N)]).reshape(E_ROWS, 128)
    ea2d = jnp.pad(ea, (0, pad)).reshape(E_ROWS, 128)
    x_flat = jnp.pad(x[:, 0], (0, N_PAD - N))
    x2d = x_flat.reshape(N_ROWS, 128)

    cnt1, sum1, xs2d, xd2d = _build_sc_pass1()(x_flat, src2d, dst2d, ea2d)
    cnt2d = cnt1.reshape(NC, N_ROWS, 128)
    sum2d = sum1.reshape(NC, N_ROWS, 128)

    bb = (b_l + b_r).reshape(1, HC)
    att_f = att.reshape(1, HC)
    p0, p1, q0, q1, den0, num0 = _tc_pass(
        xs2d, xd2d, ea2d, x2d, cnt2d, sum2d, W_l, W_r, W_e, bb, att_f)

    accs = _build_sc_pass2()(dst2d, p0, p1, q0, q1)
    c0, c1, c2, c3 = (a.reshape(NC, N_ROWS, 128) for a in accs)

    wb = (b_l + bias).reshape(1, HC)
    outp = _tc_final(c0, c1, c2, c3, den0, num0, message, W_fc,
                     b_fc.reshape(1, HC), W_l, wb)
    return outp[:N]
